# baseline (speedup 1.0000x reference)
import functools

import jax
import jax.numpy as jnp
from jax.experimental import pallas as pl


def _prep_kernel(x_ref, wk_ref, wq_ref, w0_ref):
    xb = x_ref[...]
    wk = wk_ref[...]
    wq = wq_ref[...]
    cols = []
    for b in range(xb.shape[0]):
        kb = jax.lax.dot_general(xb[b], wk, (((1,), (1,)), ((), ())),
                                 preferred_element_type=jnp.float32)
        qb = jax.lax.dot_general(xb[b], wq, (((1,), (1,)), ((), ())),
                                 preferred_element_type=jnp.float32)
        cols.append(jnp.mean(kb * qb, axis=-1, keepdims=True))
    w0_ref[...] = jnp.concatenate(cols, axis=1)


def _matvec_kernel(c_ref, w_ref, y_ref, *, decay):
    y_ref[...] = jnp.dot(c_ref[...], w_ref[...] * decay,
                         preferred_element_type=jnp.float32)


def _final_kernel(x_ref, wv_ref, w0_ref, y1_ref, y2_ref, g_ref, bb_ref,
                  out_ref, *, eps):
    xb = x_ref[...]
    wv = wv_ref[...]
    w = w0_ref[...] + y1_ref[...] + y2_ref[...]
    g = g_ref[...]
    beta = bb_ref[...]
    for b in range(xb.shape[0]):
        vb = jax.lax.dot_general(xb[b], wv, (((1,), (1,)), ((), ())),
                                 preferred_element_type=jnp.float32)
        ob = vb * w[:, b:b + 1]
        mu = jnp.mean(ob, axis=-1, keepdims=True)
        var = jnp.mean((ob - mu) ** 2, axis=-1, keepdims=True)
        out_ref[b] = (ob - mu) / jnp.sqrt(var + eps) * g + beta


def kernel(x, connection_matrix, Wk, Wq, Wv, gamma, beta):
    B, N, Cin = x.shape
    KD = Wk.shape[0]
    Cout = Wv.shape[0]
    decay = 0.7
    eps = 1e-5

    BM1 = 1000
    w0 = pl.pallas_call(
        _prep_kernel,
        grid=(N // BM1,),
        in_specs=[
            pl.BlockSpec((B, BM1, Cin), lambda i: (0, i, 0)),
            pl.BlockSpec((KD, Cin), lambda i: (0, 0)),
            pl.BlockSpec((KD, Cin), lambda i: (0, 0)),
        ],
        out_specs=pl.BlockSpec((BM1, B), lambda i: (i, 0)),
        out_shape=jax.ShapeDtypeStruct((N, B), jnp.float32),
    )(x, Wk, Wq)

    BM2 = 400
    mv = pl.pallas_call(
        functools.partial(_matvec_kernel, decay=decay),
        grid=(N // BM2,),
        in_specs=[
            pl.BlockSpec((BM2, N), lambda i: (i, 0)),
            pl.BlockSpec((N, B), lambda i: (0, 0)),
        ],
        out_specs=pl.BlockSpec((BM2, B), lambda i: (i, 0)),
        out_shape=jax.ShapeDtypeStruct((N, B), jnp.float32),
    )
    y1 = mv(connection_matrix, w0)
    y2 = mv(connection_matrix, y1)

    BM4 = 1000
    out = pl.pallas_call(
        functools.partial(_final_kernel, eps=eps),
        grid=(N // BM4,),
        in_specs=[
            pl.BlockSpec((B, BM4, Cin), lambda i: (0, i, 0)),
            pl.BlockSpec((Cout, Cin), lambda i: (0, 0)),
            pl.BlockSpec((BM4, B), lambda i: (i, 0)),
            pl.BlockSpec((BM4, B), lambda i: (i, 0)),
            pl.BlockSpec((BM4, B), lambda i: (i, 0)),
            pl.BlockSpec((1, Cout), lambda i: (0, 0)),
            pl.BlockSpec((1, Cout), lambda i: (0, 0)),
        ],
        out_specs=pl.BlockSpec((B, BM4, Cout), lambda i: (0, i, 0)),
        out_shape=jax.ShapeDtypeStruct((B, N, Cout), jnp.float32),
    )(x, Wv, w0, y1, y2, gamma.reshape(1, Cout), beta.reshape(1, Cout))

    return (out, connection_matrix)


# speedup vs baseline: 1.2817x; 1.2817x over previous
import functools

import jax
import jax.numpy as jnp
from jax.experimental import pallas as pl


def _prep_kernel(x_ref, wk_ref, wq_ref, w0_ref):
    xb = x_ref[...]
    wk = wk_ref[...]
    wq = wq_ref[...]
    cols = []
    for b in range(xb.shape[0]):
        kb = jax.lax.dot_general(xb[b], wk, (((1,), (1,)), ((), ())),
                                 preferred_element_type=jnp.float32)
        qb = jax.lax.dot_general(xb[b], wq, (((1,), (1,)), ((), ())),
                                 preferred_element_type=jnp.float32)
        cols.append(jnp.mean(kb * qb, axis=-1, keepdims=True))
    w0_ref[...] = jnp.concatenate(cols, axis=1)


def _matvec_kernel(c_ref, w_ref, y_ref, *, decay):
    y_ref[...] = jnp.dot(c_ref[...], w_ref[...] * decay,
                         preferred_element_type=jnp.float32)


def _matvec_copy_kernel(c_ref, w_ref, y_ref, cc_ref, *, decay):
    cb = c_ref[...]
    y_ref[...] = jnp.dot(cb, w_ref[...] * decay,
                         preferred_element_type=jnp.float32)
    cc_ref[...] = cb


def _final_kernel(x_ref, wv_ref, w0_ref, y1_ref, y2_ref, g_ref, bb_ref,
                  out_ref, *, eps):
    xb = x_ref[...]
    wv = wv_ref[...]
    w = w0_ref[...] + y1_ref[...] + y2_ref[...]
    g = g_ref[...]
    beta = bb_ref[...]
    for b in range(xb.shape[0]):
        vb = jax.lax.dot_general(xb[b], wv, (((1,), (1,)), ((), ())),
                                 preferred_element_type=jnp.float32)
        ob = vb * w[:, b:b + 1]
        mu = jnp.mean(ob, axis=-1, keepdims=True)
        var = jnp.mean((ob - mu) ** 2, axis=-1, keepdims=True)
        out_ref[b] = (ob - mu) / jnp.sqrt(var + eps) * g + beta


def kernel(x, connection_matrix, Wk, Wq, Wv, gamma, beta):
    B, N, Cin = x.shape
    KD = Wk.shape[0]
    Cout = Wv.shape[0]
    decay = 0.7
    eps = 1e-5

    BM1 = 1000
    w0 = pl.pallas_call(
        _prep_kernel,
        grid=(N // BM1,),
        in_specs=[
            pl.BlockSpec((B, BM1, Cin), lambda i: (0, i, 0)),
            pl.BlockSpec((KD, Cin), lambda i: (0, 0)),
            pl.BlockSpec((KD, Cin), lambda i: (0, 0)),
        ],
        out_specs=pl.BlockSpec((BM1, B), lambda i: (i, 0)),
        out_shape=jax.ShapeDtypeStruct((N, B), jnp.float32),
    )(x, Wk, Wq)

    BM2 = 200
    mv_copy = pl.pallas_call(
        functools.partial(_matvec_copy_kernel, decay=decay),
        grid=(N // BM2,),
        in_specs=[
            pl.BlockSpec((BM2, N), lambda i: (i, 0)),
            pl.BlockSpec((N, B), lambda i: (0, 0)),
        ],
        out_specs=[
            pl.BlockSpec((BM2, B), lambda i: (i, 0)),
            pl.BlockSpec((BM2, N), lambda i: (i, 0)),
        ],
        out_shape=[
            jax.ShapeDtypeStruct((N, B), jnp.float32),
            jax.ShapeDtypeStruct((N, N), jnp.float32),
        ],
    )
    mv = pl.pallas_call(
        functools.partial(_matvec_kernel, decay=decay),
        grid=(N // BM2,),
        in_specs=[
            pl.BlockSpec((BM2, N), lambda i: (i, 0)),
            pl.BlockSpec((N, B), lambda i: (0, 0)),
        ],
        out_specs=pl.BlockSpec((BM2, B), lambda i: (i, 0)),
        out_shape=jax.ShapeDtypeStruct((N, B), jnp.float32),
    )
    y1, c_copy = mv_copy(connection_matrix, w0)
    y2 = mv(connection_matrix, y1)

    BM4 = 1000
    out = pl.pallas_call(
        functools.partial(_final_kernel, eps=eps),
        grid=(N // BM4,),
        in_specs=[
            pl.BlockSpec((B, BM4, Cin), lambda i: (0, i, 0)),
            pl.BlockSpec((Cout, Cin), lambda i: (0, 0)),
            pl.BlockSpec((BM4, B), lambda i: (i, 0)),
            pl.BlockSpec((BM4, B), lambda i: (i, 0)),
            pl.BlockSpec((BM4, B), lambda i: (i, 0)),
            pl.BlockSpec((1, Cout), lambda i: (0, 0)),
            pl.BlockSpec((1, Cout), lambda i: (0, 0)),
        ],
        out_specs=pl.BlockSpec((B, BM4, Cout), lambda i: (0, i, 0)),
        out_shape=jax.ShapeDtypeStruct((B, N, Cout), jnp.float32),
    )(x, Wv, w0, y1, y2, gamma.reshape(1, Cout), beta.reshape(1, Cout))

    return (out, c_copy)


# speedup vs baseline: 1.3196x; 1.0296x over previous
import functools

import jax
import jax.numpy as jnp
from jax.experimental import pallas as pl


def _prep_kernel(x_ref, wk_ref, wq_ref, w0_ref):
    xb = x_ref[...]
    wk = wk_ref[...]
    wq = wq_ref[...]
    cols = []
    for b in range(xb.shape[0]):
        kb = jax.lax.dot_general(xb[b], wk, (((1,), (1,)), ((), ())),
                                 preferred_element_type=jnp.float32)
        qb = jax.lax.dot_general(xb[b], wq, (((1,), (1,)), ((), ())),
                                 preferred_element_type=jnp.float32)
        cols.append(jnp.mean(kb * qb, axis=-1, keepdims=True))
    w0_ref[...] = jnp.concatenate(cols, axis=1)


def _matvec_kernel(c_ref, w_ref, y_ref, *, decay):
    y_ref[...] = jnp.dot(c_ref[...], w_ref[...] * decay,
                         preferred_element_type=jnp.float32)


def _matvec_copy_kernel(c_ref, w_ref, y_ref, cc_ref, *, decay):
    cb = c_ref[...]
    y_ref[...] = jnp.dot(cb, w_ref[...] * decay,
                         preferred_element_type=jnp.float32)
    cc_ref[...] = cb


def _mv2_final_kernel(c_ref, w0_ref, y1_ref, x_ref, wv_ref, g_ref, bb_ref,
                      out_ref, *, decay, eps):
    i = pl.program_id(0)
    bm = c_ref.shape[0]
    y1 = y1_ref[...]
    y2 = jnp.dot(c_ref[...], y1 * decay,
                 preferred_element_type=jnp.float32)
    wtot = w0_ref[...] + y1_ref[pl.ds(i * bm, bm), :] + y2
    xb = x_ref[...]
    wv = wv_ref[...]
    g = g_ref[...]
    beta = bb_ref[...]
    for b in range(xb.shape[0]):
        vb = jax.lax.dot_general(xb[b], wv, (((1,), (1,)), ((), ())),
                                 preferred_element_type=jnp.float32)
        ob = vb * wtot[:, b:b + 1]
        mu = jnp.mean(ob, axis=-1, keepdims=True)
        var = jnp.mean((ob - mu) ** 2, axis=-1, keepdims=True)
        out_ref[b] = (ob - mu) / jnp.sqrt(var + eps) * g + beta


def kernel(x, connection_matrix, Wk, Wq, Wv, gamma, beta):
    B, N, Cin = x.shape
    KD = Wk.shape[0]
    Cout = Wv.shape[0]
    decay = 0.7
    eps = 1e-5

    BM1 = 1000
    w0 = pl.pallas_call(
        _prep_kernel,
        grid=(N // BM1,),
        in_specs=[
            pl.BlockSpec((B, BM1, Cin), lambda i: (0, i, 0)),
            pl.BlockSpec((KD, Cin), lambda i: (0, 0)),
            pl.BlockSpec((KD, Cin), lambda i: (0, 0)),
        ],
        out_specs=pl.BlockSpec((BM1, B), lambda i: (i, 0)),
        out_shape=jax.ShapeDtypeStruct((N, B), jnp.float32),
    )(x, Wk, Wq)

    BM2 = 200
    mv_copy = pl.pallas_call(
        functools.partial(_matvec_copy_kernel, decay=decay),
        grid=(N // BM2,),
        in_specs=[
            pl.BlockSpec((BM2, N), lambda i: (i, 0)),
            pl.BlockSpec((N, B), lambda i: (0, 0)),
        ],
        out_specs=[
            pl.BlockSpec((BM2, B), lambda i: (i, 0)),
            pl.BlockSpec((BM2, N), lambda i: (i, 0)),
        ],
        out_shape=[
            jax.ShapeDtypeStruct((N, B), jnp.float32),
            jax.ShapeDtypeStruct((N, N), jnp.float32),
        ],
    )
    y1, c_copy = mv_copy(connection_matrix, w0)

    out = pl.pallas_call(
        functools.partial(_mv2_final_kernel, decay=decay, eps=eps),
        grid=(N // BM2,),
        in_specs=[
            pl.BlockSpec((BM2, N), lambda i: (i, 0)),
            pl.BlockSpec((BM2, B), lambda i: (i, 0)),
            pl.BlockSpec((N, B), lambda i: (0, 0)),
            pl.BlockSpec((B, BM2, Cin), lambda i: (0, i, 0)),
            pl.BlockSpec((Cout, Cin), lambda i: (0, 0)),
            pl.BlockSpec((1, Cout), lambda i: (0, 0)),
            pl.BlockSpec((1, Cout), lambda i: (0, 0)),
        ],
        out_specs=pl.BlockSpec((B, BM2, Cout), lambda i: (0, i, 0)),
        out_shape=jax.ShapeDtypeStruct((B, N, Cout), jnp.float32),
    )(connection_matrix, w0, y1, x, Wv,
      gamma.reshape(1, Cout), beta.reshape(1, Cout))

    return (out, c_copy)
